# baseline (speedup 1.0000x reference)
import jax
import jax.numpy as jnp
from jax import lax
from jax.experimental import pallas as pl
from jax.experimental.pallas import tpu as pltpu
from jax.experimental.pallas import tpu_sc as plsc

_NC = 2
_NS = 16
_NW = _NC * _NS
_K = 50
_NB = 4
_IR = 8


def _sc_mesh():
    return plsc.VectorSubcoreMesh(core_axis_name="c", subcore_axis_name="s",
                                  num_cores=_NC, num_subcores=_NS)


def _sc_deg(idx_rs, zeros_n, ones_k, n):
    nch = idx_rs.shape[1]

    def body(idx_hbm, zeros_hbm, ones_hbm, out_hbm, acc, idx_all, ones_v,
             dsem):
        cid = lax.axis_index("c")
        sid = lax.axis_index("s")
        wid = cid * _NS + sid

        @pl.when(sid == 0)
        def _():
            pltpu.sync_copy(zeros_hbm, acc)

        pltpu.sync_copy(idx_hbm.at[wid], idx_all)
        pltpu.sync_copy(ones_hbm, ones_v)
        plsc.subcore_barrier()

        def chunk(i, carry):
            pltpu.async_copy(ones_v, acc.at[idx_all.at[i, 1]], dsem,
                             add=True)
            return carry

        lax.fori_loop(0, nch, chunk, 0)

        def drain(i, carry):
            pltpu.make_async_copy(ones_v, acc.at[idx_all.at[0, 1]],
                                  dsem).wait()
            return carry

        lax.fori_loop(0, nch, drain, 0)
        plsc.subcore_barrier()

        @pl.when(sid == 0)
        def _():
            pltpu.sync_copy(acc, out_hbm.at[cid, 0])

    f = pl.kernel(
        body,
        out_type=jax.ShapeDtypeStruct((_NC, 1, n), jnp.float32),
        mesh=_sc_mesh(),
        scratch_types=[
            pltpu.VMEM_SHARED((n,), jnp.float32),
            pltpu.VMEM((nch, 2, _K), jnp.int32),
            pltpu.VMEM((_K,), jnp.float32),
            pltpu.SemaphoreType.DMA,
        ],
    )
    return f(idx_rs, zeros_n, ones_k)


def _sc_edges(g, idx_rs, zeros_rt):
    n, d = g.shape
    nch = idx_rs.shape[1]
    rt = (n // _NS) // 8 * 8
    tail = n - rt * _NS

    def body(g_hbm, idx_hbm, zeros_hbm, out_hbm, acc, idxb, rows,
             gsem, isem, ssem):
        cid = lax.axis_index("c")
        sid = lax.axis_index("s")
        wid = cid * _NS + sid

        def idx_start(ch, slot):
            pltpu.async_copy(idx_hbm.at[wid, ch], idxb.at[slot],
                             isem.at[slot])

        def idx_wait(slot):
            pltpu.make_async_copy(idx_hbm.at[wid, 0], idxb.at[slot],
                                  isem.at[slot]).wait()

        def gather_start(islot, rslot):
            pltpu.async_copy(g_hbm.at[idxb.at[islot, 0]], rows.at[rslot],
                             gsem.at[rslot])

        def gather_wait(rslot):
            pltpu.make_async_copy(g_hbm.at[idxb.at[0, 0]], rows.at[rslot],
                                  gsem.at[rslot]).wait()

        def scatter_start(islot, rslot):
            pltpu.async_copy(rows.at[rslot], acc.at[idxb.at[islot, 1]],
                             ssem.at[rslot], add=True)

        def scatter_wait(rslot):
            pltpu.make_async_copy(rows.at[rslot], acc.at[idxb.at[0, 1]],
                                  ssem.at[rslot]).wait()

        def chunk_ops(i, b, first=False, do_refill=True, do_gather=True):
            r = b % _NB
            rw = (r + 2) % _NB
            bq = (b + _IR - 2) % _IR
            bg = (b + 2) % _IR
            gather_wait(r)
            if not first:
                scatter_wait(rw)
            scatter_start(b, r)
            if do_refill:
                idx_start(i + _IR - 2, bq)
            if do_gather:
                idx_wait(bg)
                gather_start(bg, rw)

        for s in range(_IR - 2):
            idx_start(s, s)
        for b in range(2):
            idx_wait(b)
            gather_start(b, b)

        pltpu.sync_copy(zeros_hbm, acc.at[pl.ds(sid * rt, rt)])

        @pl.when(sid == 0)
        def _():
            pltpu.sync_copy(zeros_hbm.at[pl.ds(0, tail)],
                            acc.at[pl.ds(rt * _NS, tail)])

        plsc.subcore_barrier()

        for b in range(_IR):
            chunk_ops(b, b, first=(b < 2))

        def outer(gi, carry):
            for b in range(_IR):
                chunk_ops(gi * _IR + b, b)
            return carry

        lax.fori_loop(1, nch // _IR - 1, outer, 0)

        for b in range(_IR):
            chunk_ops(nch - _IR + b, b, do_refill=(b < 2),
                      do_gather=(b < _IR - 2))
        scatter_wait((nch - 2) % _NB)
        scatter_wait((nch - 1) % _NB)

        plsc.subcore_barrier()
        pltpu.sync_copy(acc.at[pl.ds(sid * rt, rt)],
                        out_hbm.at[cid, pl.ds(sid * rt, rt)])

        @pl.when(sid == 0)
        def _():
            pltpu.sync_copy(acc.at[pl.ds(rt * _NS, tail)],
                            out_hbm.at[cid, pl.ds(rt * _NS, tail)])

    f = pl.kernel(
        body,
        out_type=jax.ShapeDtypeStruct((_NC, n, d), jnp.float32),
        mesh=_sc_mesh(),
        scratch_types=[
            pltpu.VMEM_SHARED((n, d), jnp.float32),
            pltpu.VMEM((_IR, 2, _K), jnp.int32),
            pltpu.VMEM((_NB, _K, d), jnp.float32),
            pltpu.SemaphoreType.DMA((_NB,)),
            pltpu.SemaphoreType.DMA((_IR,)),
            pltpu.SemaphoreType.DMA((_NB,)),
        ],
    )
    return f(g, idx_rs, zeros_rt)


def _tc_g1(degT, x, w1, bsz):
    n, d = x.shape

    def body(degT_ref, x_ref, w_ref, g_ref, dis_ref):
        deg = degT_ref[:, 0:1] + degT_ref[:, 1:2] + 1.0
        dis = lax.rsqrt(deg)
        h = jnp.dot(x_ref[...], w_ref[...], preferred_element_type=jnp.float32)
        g_ref[...] = h * dis
        dis_ref[...] = dis

    return pl.pallas_call(
        body,
        grid=(n // bsz,),
        in_specs=[
            pl.BlockSpec((bsz, 2), lambda i: (i, 0)),
            pl.BlockSpec((bsz, d), lambda i: (i, 0)),
            pl.BlockSpec((d, d), lambda i: (0, 0)),
        ],
        out_specs=[
            pl.BlockSpec((bsz, d), lambda i: (i, 0)),
            pl.BlockSpec((bsz, 1), lambda i: (i, 0)),
        ],
        out_shape=[
            jax.ShapeDtypeStruct((n, d), jnp.float32),
            jax.ShapeDtypeStruct((n, 1), jnp.float32),
        ],
    )(degT, x, w1)


def _tc_layer(sp, g, dis, b, w, bsz):
    n, d = g.shape

    def body(sp_ref, g_ref, dis_ref, b_ref, w_ref, out_ref):
        s = sp_ref[0] + sp_ref[1] + g_ref[...]
        h = jnp.tanh(dis_ref[...] * s + b_ref[...])
        out_ref[...] = dis_ref[...] * jnp.dot(
            h, w_ref[...], preferred_element_type=jnp.float32)

    return pl.pallas_call(
        body,
        grid=(n // bsz,),
        in_specs=[
            pl.BlockSpec((2, bsz, d), lambda i: (0, i, 0)),
            pl.BlockSpec((bsz, d), lambda i: (i, 0)),
            pl.BlockSpec((bsz, 1), lambda i: (i, 0)),
            pl.BlockSpec((1, d), lambda i: (0, 0)),
            pl.BlockSpec((d, d), lambda i: (0, 0)),
        ],
        out_specs=pl.BlockSpec((bsz, d), lambda i: (i, 0)),
        out_shape=jax.ShapeDtypeStruct((n, d), jnp.float32),
    )(sp, g, dis, b, w)


def _tc_head(sp, g, dis, b2, wf1, bf1, wf2, bf2, bsz):
    n, d = g.shape
    h_fc = wf1.shape[1]
    n_out = wf2.shape[1]

    def body(sp_ref, g_ref, dis_ref, b2_ref, wf1_ref, bf1_ref, wf2_ref,
             bf2_ref, out_ref):
        s = sp_ref[0] + sp_ref[1] + g_ref[...]
        h = jnp.tanh(dis_ref[...] * s + b2_ref[...])
        f = jnp.tanh(jnp.dot(h, wf1_ref[...],
                             preferred_element_type=jnp.float32) + bf1_ref[...])
        out_ref[...] = jnp.dot(
            f, wf2_ref[...], preferred_element_type=jnp.float32) + bf2_ref[...]

    return pl.pallas_call(
        body,
        grid=(n // bsz,),
        in_specs=[
            pl.BlockSpec((2, bsz, d), lambda i: (0, i, 0)),
            pl.BlockSpec((bsz, d), lambda i: (i, 0)),
            pl.BlockSpec((bsz, 1), lambda i: (i, 0)),
            pl.BlockSpec((1, d), lambda i: (0, 0)),
            pl.BlockSpec((d, h_fc), lambda i: (0, 0)),
            pl.BlockSpec((1, h_fc), lambda i: (0, 0)),
            pl.BlockSpec((h_fc, n_out), lambda i: (0, 0)),
            pl.BlockSpec((1, n_out), lambda i: (0, 0)),
        ],
        out_specs=pl.BlockSpec((bsz, n_out), lambda i: (i, 0)),
        out_shape=jax.ShapeDtypeStruct((n, n_out), jnp.float32),
    )(sp, g, dis, b2, wf1, bf1, wf2, bf2)


def kernel(x, edge_index, W1, b1, W2, b2, Wf1, bf1, Wf2, bf2):
    n, d = x.shape
    e = edge_index.shape[1]
    ew = e // _NW
    nch = ew // _K
    idx_rs = edge_index.reshape(2, _NW, nch, _K).transpose(1, 2, 0, 3)

    zeros_n = jnp.zeros((n,), jnp.float32)
    ones_k = jnp.ones((_K,), jnp.float32)
    zeros_rt = jnp.zeros(((n // _NS) // 8 * 8, d), jnp.float32)

    bsz = 2000

    degp = _sc_deg(idx_rs, zeros_n, ones_k, n)
    degT = degp.reshape(_NC, n).T
    g1, dis = _tc_g1(degT, x, W1, bsz)
    s1p = _sc_edges(g1, idx_rs, zeros_rt)
    g2 = _tc_layer(s1p, g1, dis, b1.reshape(1, d), W2, bsz)
    s2p = _sc_edges(g2, idx_rs, zeros_rt)
    out = _tc_head(s2p, g2, dis, b2.reshape(1, d), Wf1,
                   bf1.reshape(1, -1), Wf2, bf2.reshape(1, -1), bsz)
    return out


# speedup vs baseline: 1.1365x; 1.1365x over previous
import jax
import jax.numpy as jnp
from jax import lax
from jax.experimental import pallas as pl
from jax.experimental.pallas import tpu as pltpu
from jax.experimental.pallas import tpu_sc as plsc

_NC = 2
_NS = 16
_NW = _NC * _NS
_K = 50
_NB = 5
_IR = 8
_BLK = 40


def _sc_mesh():
    return plsc.VectorSubcoreMesh(core_axis_name="c", subcore_axis_name="s",
                                  num_cores=_NC, num_subcores=_NS)


def _sc_deg(idx_rs, zeros_n, ones_k, n):
    nch = idx_rs.shape[1]

    def body(idx_hbm, zeros_hbm, ones_hbm, out_hbm, acc, idx_all, ones_v,
             dsem):
        cid = lax.axis_index("c")
        sid = lax.axis_index("s")
        wid = cid * _NS + sid

        @pl.when(sid == 0)
        def _():
            pltpu.sync_copy(zeros_hbm, acc)

        pltpu.sync_copy(idx_hbm.at[wid], idx_all)
        pltpu.sync_copy(ones_hbm, ones_v)
        plsc.subcore_barrier()

        def chunk(i, carry):
            pltpu.async_copy(ones_v, acc.at[idx_all.at[i, 1]], dsem,
                             add=True)
            return carry

        lax.fori_loop(0, nch, chunk, 0)

        def drain(i, carry):
            pltpu.make_async_copy(ones_v, acc.at[idx_all.at[0, 1]],
                                  dsem).wait()
            return carry

        lax.fori_loop(0, nch, drain, 0)
        plsc.subcore_barrier()

        @pl.when(sid == 0)
        def _():
            pltpu.sync_copy(acc, out_hbm.at[cid, 0])

    f = pl.kernel(
        body,
        out_type=jax.ShapeDtypeStruct((_NC, 1, n), jnp.float32),
        mesh=_sc_mesh(),
        scratch_types=[
            pltpu.VMEM_SHARED((n,), jnp.float32),
            pltpu.VMEM((nch, 2, _K), jnp.int32),
            pltpu.VMEM((_K,), jnp.float32),
            pltpu.SemaphoreType.DMA,
        ],
    )
    return f(idx_rs, zeros_n, ones_k)


def _sc_edges(g, idx_rs, zeros_rt):
    n, d = g.shape
    nch = idx_rs.shape[1]
    rt = (n // _NS) // 8 * 8
    tail = n - rt * _NS

    def body(g_hbm, idx_hbm, zeros_hbm, out_hbm, acc, idxb, rows,
             gsem, isem, ssem):
        cid = lax.axis_index("c")
        sid = lax.axis_index("s")
        wid = cid * _NS + sid

        def idx_start(ch, slot):
            pltpu.async_copy(idx_hbm.at[wid, ch], idxb.at[slot],
                             isem.at[slot])

        def idx_wait(slot):
            pltpu.make_async_copy(idx_hbm.at[wid, 0], idxb.at[slot],
                                  isem.at[slot]).wait()

        def gather_start(islot, rslot):
            pltpu.async_copy(g_hbm.at[idxb.at[islot, 0]], rows.at[rslot],
                             gsem.at[rslot])

        def gather_wait(rslot):
            pltpu.make_async_copy(g_hbm.at[idxb.at[0, 0]], rows.at[rslot],
                                  gsem.at[rslot]).wait()

        def scatter_start(islot, rslot):
            pltpu.async_copy(rows.at[rslot], acc.at[idxb.at[islot, 1]],
                             ssem.at[rslot], add=True)

        def scatter_wait(rslot):
            pltpu.make_async_copy(rows.at[rslot], acc.at[idxb.at[0, 1]],
                                  ssem.at[rslot]).wait()

        pltpu.sync_copy(zeros_hbm, acc.at[pl.ds(sid * rt, rt)])

        @pl.when(sid == 0)
        def _():
            pltpu.sync_copy(zeros_hbm.at[pl.ds(0, tail)],
                            acc.at[pl.ds(rt * _NS, tail)])

        plsc.subcore_barrier()

        def chunk_ops(i, j, skip_sw=False, do_refill=True, do_gather=True):
            b = j % _IR
            r = j % _NB
            rw = (r + 3) % _NB
            bq = (b + _IR - 2) % _IR
            bg = (b + 3) % _IR
            gather_wait(r)
            if not skip_sw:
                scatter_wait(rw)
            scatter_start(b, r)
            if do_refill:
                idx_start(i + _IR - 2, bq)
            if do_gather:
                idx_wait(bg)
                gather_start(bg, rw)

        for s in range(_IR - 2):
            idx_start(s, s)
        for b in range(3):
            idx_wait(b)
            gather_start(b, b)

        for j in range(_BLK):
            chunk_ops(j, j, skip_sw=(j < 2))

        def outer(gi, carry):
            for j in range(_BLK):
                chunk_ops(gi * _BLK + j, j)
            return carry

        lax.fori_loop(1, nch // _BLK - 1, outer, 0)

        for j in range(_BLK):
            chunk_ops(nch - _BLK + j, j, do_refill=(j < _BLK - _IR + 2),
                      do_gather=(j < _BLK - 3))
        scatter_wait((nch - 2) % _NB)
        scatter_wait((nch - 1) % _NB)

        plsc.subcore_barrier()
        pltpu.sync_copy(acc.at[pl.ds(sid * rt, rt)],
                        out_hbm.at[cid, pl.ds(sid * rt, rt)])

        @pl.when(sid == 0)
        def _():
            pltpu.sync_copy(acc.at[pl.ds(rt * _NS, tail)],
                            out_hbm.at[cid, pl.ds(rt * _NS, tail)])

    f = pl.kernel(
        body,
        out_type=jax.ShapeDtypeStruct((_NC, n, d), jnp.float32),
        mesh=_sc_mesh(),
        scratch_types=[
            pltpu.VMEM_SHARED((n, d), jnp.float32),
            pltpu.VMEM((_IR, 2, _K), jnp.int32),
            pltpu.VMEM((_NB, _K, d), jnp.float32),
            pltpu.SemaphoreType.DMA((_NB,)),
            pltpu.SemaphoreType.DMA((_IR,)),
            pltpu.SemaphoreType.DMA((_NB,)),
        ],
    )
    return f(g, idx_rs, zeros_rt)


def _tc_g1(degT, x, w1, bsz):
    n, d = x.shape

    def body(degT_ref, x_ref, w_ref, g_ref, dis_ref):
        deg = degT_ref[:, 0:1] + degT_ref[:, 1:2] + 1.0
        dis = lax.rsqrt(deg)
        h = jnp.dot(x_ref[...], w_ref[...], preferred_element_type=jnp.float32)
        g_ref[...] = h * dis
        dis_ref[...] = dis

    return pl.pallas_call(
        body,
        grid=(n // bsz,),
        in_specs=[
            pl.BlockSpec((bsz, 2), lambda i: (i, 0)),
            pl.BlockSpec((bsz, d), lambda i: (i, 0)),
            pl.BlockSpec((d, d), lambda i: (0, 0)),
        ],
        out_specs=[
            pl.BlockSpec((bsz, d), lambda i: (i, 0)),
            pl.BlockSpec((bsz, 1), lambda i: (i, 0)),
        ],
        out_shape=[
            jax.ShapeDtypeStruct((n, d), jnp.float32),
            jax.ShapeDtypeStruct((n, 1), jnp.float32),
        ],
    )(degT, x, w1)


def _tc_layer(sp, g, dis, b, w, bsz):
    n, d = g.shape

    def body(sp_ref, g_ref, dis_ref, b_ref, w_ref, out_ref):
        s = sp_ref[0] + sp_ref[1] + g_ref[...]
        h = jnp.tanh(dis_ref[...] * s + b_ref[...])
        out_ref[...] = dis_ref[...] * jnp.dot(
            h, w_ref[...], preferred_element_type=jnp.float32)

    return pl.pallas_call(
        body,
        grid=(n // bsz,),
        in_specs=[
            pl.BlockSpec((2, bsz, d), lambda i: (0, i, 0)),
            pl.BlockSpec((bsz, d), lambda i: (i, 0)),
            pl.BlockSpec((bsz, 1), lambda i: (i, 0)),
            pl.BlockSpec((1, d), lambda i: (0, 0)),
            pl.BlockSpec((d, d), lambda i: (0, 0)),
        ],
        out_specs=pl.BlockSpec((bsz, d), lambda i: (i, 0)),
        out_shape=jax.ShapeDtypeStruct((n, d), jnp.float32),
    )(sp, g, dis, b, w)


def _tc_head(sp, g, dis, b2, wf1, bf1, wf2, bf2, bsz):
    n, d = g.shape
    h_fc = wf1.shape[1]
    n_out = wf2.shape[1]

    def body(sp_ref, g_ref, dis_ref, b2_ref, wf1_ref, bf1_ref, wf2_ref,
             bf2_ref, out_ref):
        s = sp_ref[0] + sp_ref[1] + g_ref[...]
        h = jnp.tanh(dis_ref[...] * s + b2_ref[...])
        f = jnp.tanh(jnp.dot(h, wf1_ref[...],
                             preferred_element_type=jnp.float32) + bf1_ref[...])
        out_ref[...] = jnp.dot(
            f, wf2_ref[...], preferred_element_type=jnp.float32) + bf2_ref[...]

    return pl.pallas_call(
        body,
        grid=(n // bsz,),
        in_specs=[
            pl.BlockSpec((2, bsz, d), lambda i: (0, i, 0)),
            pl.BlockSpec((bsz, d), lambda i: (i, 0)),
            pl.BlockSpec((bsz, 1), lambda i: (i, 0)),
            pl.BlockSpec((1, d), lambda i: (0, 0)),
            pl.BlockSpec((d, h_fc), lambda i: (0, 0)),
            pl.BlockSpec((1, h_fc), lambda i: (0, 0)),
            pl.BlockSpec((h_fc, n_out), lambda i: (0, 0)),
            pl.BlockSpec((1, n_out), lambda i: (0, 0)),
        ],
        out_specs=pl.BlockSpec((bsz, n_out), lambda i: (i, 0)),
        out_shape=jax.ShapeDtypeStruct((n, n_out), jnp.float32),
    )(sp, g, dis, b2, wf1, bf1, wf2, bf2)


def kernel(x, edge_index, W1, b1, W2, b2, Wf1, bf1, Wf2, bf2):
    n, d = x.shape
    e = edge_index.shape[1]
    ew = e // _NW
    nch = ew // _K
    idx_rs = edge_index.reshape(2, _NW, nch, _K).transpose(1, 2, 0, 3)

    zeros_n = jnp.zeros((n,), jnp.float32)
    ones_k = jnp.ones((_K,), jnp.float32)
    zeros_rt = jnp.zeros(((n // _NS) // 8 * 8, d), jnp.float32)

    bsz = 2000

    degp = _sc_deg(idx_rs, zeros_n, ones_k, n)
    degT = degp.reshape(_NC, n).T
    g1, dis = _tc_g1(degT, x, W1, bsz)
    s1p = _sc_edges(g1, idx_rs, zeros_rt)
    g2 = _tc_layer(s1p, g1, dis, b1.reshape(1, d), W2, bsz)
    s2p = _sc_edges(g2, idx_rs, zeros_rt)
    out = _tc_head(s2p, g2, dis, b2.reshape(1, d), Wf1,
                   bf1.reshape(1, -1), Wf2, bf2.reshape(1, -1), bsz)
    return out


# speedup vs baseline: 1.2257x; 1.0785x over previous
import jax
import jax.numpy as jnp
from jax import lax
from jax.experimental import pallas as pl
from jax.experimental.pallas import tpu as pltpu
from jax.experimental.pallas import tpu_sc as plsc

_NC = 2
_NS = 16
_NW = _NC * _NS
_K = 50
_NB = 4
_IR = 8


def _sc_mesh():
    return plsc.VectorSubcoreMesh(core_axis_name="c", subcore_axis_name="s",
                                  num_cores=_NC, num_subcores=_NS)


def _sc_deg(idx_rs, zeros_n, ones_k, n):
    nch = idx_rs.shape[1]

    def body(idx_hbm, zeros_hbm, ones_hbm, out_hbm, acc, idx_all, ones_v,
             dsem):
        cid = lax.axis_index("c")
        sid = lax.axis_index("s")
        wid = cid * _NS + sid

        @pl.when(sid == 0)
        def _():
            pltpu.sync_copy(zeros_hbm, acc)

        pltpu.sync_copy(idx_hbm.at[wid], idx_all)
        pltpu.sync_copy(ones_hbm, ones_v)
        plsc.subcore_barrier()

        def chunk(i, carry):
            pltpu.async_copy(ones_v, acc.at[idx_all.at[i, 1]], dsem,
                             add=True)
            return carry

        lax.fori_loop(0, nch, chunk, 0)

        def drain(i, carry):
            pltpu.make_async_copy(ones_v, acc.at[idx_all.at[0, 1]],
                                  dsem).wait()
            return carry

        lax.fori_loop(0, nch, drain, 0)
        plsc.subcore_barrier()

        @pl.when(sid == 0)
        def _():
            pltpu.sync_copy(acc, out_hbm.at[cid, 0])

    f = pl.kernel(
        body,
        out_type=jax.ShapeDtypeStruct((_NC, 1, n), jnp.float32),
        mesh=_sc_mesh(),
        scratch_types=[
            pltpu.VMEM_SHARED((n,), jnp.float32),
            pltpu.VMEM((nch, 2, _K), jnp.int32),
            pltpu.VMEM((_K,), jnp.float32),
            pltpu.SemaphoreType.DMA,
        ],
    )
    return f(idx_rs, zeros_n, ones_k)


def _sc_edges(g, idx_rs, zeros_rt):
    n, d = g.shape
    nch = idx_rs.shape[1]
    rt = (n // _NS) // 8 * 8
    tail = n - rt * _NS

    def body(g_hbm, idx_hbm, zeros_hbm, out_hbm, acc, idxb, rows,
             gsem, isem, ssem):
        cid = lax.axis_index("c")
        sid = lax.axis_index("s")
        wid = cid * _NS + sid

        def idx_start(ch, slot):
            pltpu.async_copy(idx_hbm.at[wid, ch], idxb.at[slot],
                             isem.at[slot])

        def idx_wait(slot):
            pltpu.make_async_copy(idx_hbm.at[wid, 0], idxb.at[slot],
                                  isem.at[slot]).wait()

        def gather_start(islot, rslot):
            pltpu.async_copy(g_hbm.at[idxb.at[islot, 0]], rows.at[rslot],
                             gsem.at[rslot])

        def gather_wait(rslot):
            pltpu.make_async_copy(g_hbm.at[idxb.at[0, 0]], rows.at[rslot],
                                  gsem.at[rslot]).wait()

        def scatter_start(islot, rslot):
            pltpu.async_copy(rows.at[rslot], acc.at[idxb.at[islot, 1]],
                             ssem.at[rslot], add=True)

        def scatter_wait(rslot):
            pltpu.make_async_copy(rows.at[rslot], acc.at[idxb.at[0, 1]],
                                  ssem.at[rslot]).wait()

        for s in range(_IR - 1):
            idx_start(s, s)
        for b in range(_NB - 1):
            idx_wait(b)
            gather_start(b, b)

        pltpu.sync_copy(zeros_hbm, acc.at[pl.ds(sid * rt, rt)])

        @pl.when(sid == 0)
        def _():
            pltpu.sync_copy(zeros_hbm.at[pl.ds(0, tail)],
                            acc.at[pl.ds(rt * _NS, tail)])

        plsc.subcore_barrier()

        def chunk_ops(i, b, first=False, do_refill=True, do_gather=True):
            r = b % _NB
            rp = (r + _NB - 1) % _NB
            bp = (b + _IR - 1) % _IR
            bg = (b + _NB - 1) % _IR
            gather_wait(r)
            if not first:
                scatter_wait(rp)
            scatter_start(b, r)
            if do_refill:
                idx_start(i + _IR - 1, bp)
            if do_gather:
                idx_wait(bg)
                gather_start(bg, rp)

        for b in range(_IR):
            chunk_ops(b, b, first=(b == 0))

        def outer(gi, carry):
            for b in range(_IR):
                chunk_ops(gi * _IR + b, b)
            return carry

        lax.fori_loop(1, nch // _IR - 1, outer, 0)

        for b in range(_IR):
            chunk_ops(nch - _IR + b, b, do_refill=(b == 0),
                      do_gather=(b < _NB + 1))
        scatter_wait((nch - 1) % _NB)

        plsc.subcore_barrier()
        pltpu.sync_copy(acc.at[pl.ds(sid * rt, rt)],
                        out_hbm.at[cid, pl.ds(sid * rt, rt)])

        @pl.when(sid == 0)
        def _():
            pltpu.sync_copy(acc.at[pl.ds(rt * _NS, tail)],
                            out_hbm.at[cid, pl.ds(rt * _NS, tail)])

    f = pl.kernel(
        body,
        out_type=jax.ShapeDtypeStruct((_NC, n, d), jnp.float32),
        mesh=_sc_mesh(),
        scratch_types=[
            pltpu.VMEM_SHARED((n, d), jnp.float32),
            pltpu.VMEM((_IR, 2, _K), jnp.int32),
            pltpu.VMEM((_NB, _K, d), jnp.float32),
            pltpu.SemaphoreType.DMA((_NB,)),
            pltpu.SemaphoreType.DMA((_IR,)),
            pltpu.SemaphoreType.DMA((_NB,)),
        ],
    )
    return f(g, idx_rs, zeros_rt)


def _tc_g1(degT, x, w1, bsz):
    n, d = x.shape

    def body(degT_ref, x_ref, w_ref, g_ref, dis_ref):
        deg = degT_ref[:, 0:1] + degT_ref[:, 1:2] + 1.0
        dis = lax.rsqrt(deg)
        h = jnp.dot(x_ref[...], w_ref[...], preferred_element_type=jnp.float32)
        g_ref[...] = h * dis
        dis_ref[...] = dis

    return pl.pallas_call(
        body,
        grid=(n // bsz,),
        in_specs=[
            pl.BlockSpec((bsz, 2), lambda i: (i, 0)),
            pl.BlockSpec((bsz, d), lambda i: (i, 0)),
            pl.BlockSpec((d, d), lambda i: (0, 0)),
        ],
        out_specs=[
            pl.BlockSpec((bsz, d), lambda i: (i, 0)),
            pl.BlockSpec((bsz, 1), lambda i: (i, 0)),
        ],
        out_shape=[
            jax.ShapeDtypeStruct((n, d), jnp.float32),
            jax.ShapeDtypeStruct((n, 1), jnp.float32),
        ],
    )(degT, x, w1)


def _tc_layer(sp, g, dis, b, w, bsz):
    n, d = g.shape

    def body(sp_ref, g_ref, dis_ref, b_ref, w_ref, out_ref):
        s = sp_ref[0] + sp_ref[1] + g_ref[...]
        h = jnp.tanh(dis_ref[...] * s + b_ref[...])
        out_ref[...] = dis_ref[...] * jnp.dot(
            h, w_ref[...], preferred_element_type=jnp.float32)

    return pl.pallas_call(
        body,
        grid=(n // bsz,),
        in_specs=[
            pl.BlockSpec((2, bsz, d), lambda i: (0, i, 0)),
            pl.BlockSpec((bsz, d), lambda i: (i, 0)),
            pl.BlockSpec((bsz, 1), lambda i: (i, 0)),
            pl.BlockSpec((1, d), lambda i: (0, 0)),
            pl.BlockSpec((d, d), lambda i: (0, 0)),
        ],
        out_specs=pl.BlockSpec((bsz, d), lambda i: (i, 0)),
        out_shape=jax.ShapeDtypeStruct((n, d), jnp.float32),
    )(sp, g, dis, b, w)


def _tc_head(sp, g, dis, b2, wf1, bf1, wf2, bf2, bsz):
    n, d = g.shape
    h_fc = wf1.shape[1]
    n_out = wf2.shape[1]

    def body(sp_ref, g_ref, dis_ref, b2_ref, wf1_ref, bf1_ref, wf2_ref,
             bf2_ref, out_ref):
        s = sp_ref[0] + sp_ref[1] + g_ref[...]
        h = jnp.tanh(dis_ref[...] * s + b2_ref[...])
        f = jnp.tanh(jnp.dot(h, wf1_ref[...],
                             preferred_element_type=jnp.float32) + bf1_ref[...])
        out_ref[...] = jnp.dot(
            f, wf2_ref[...], preferred_element_type=jnp.float32) + bf2_ref[...]

    return pl.pallas_call(
        body,
        grid=(n // bsz,),
        in_specs=[
            pl.BlockSpec((2, bsz, d), lambda i: (0, i, 0)),
            pl.BlockSpec((bsz, d), lambda i: (i, 0)),
            pl.BlockSpec((bsz, 1), lambda i: (i, 0)),
            pl.BlockSpec((1, d), lambda i: (0, 0)),
            pl.BlockSpec((d, h_fc), lambda i: (0, 0)),
            pl.BlockSpec((1, h_fc), lambda i: (0, 0)),
            pl.BlockSpec((h_fc, n_out), lambda i: (0, 0)),
            pl.BlockSpec((1, n_out), lambda i: (0, 0)),
        ],
        out_specs=pl.BlockSpec((bsz, n_out), lambda i: (i, 0)),
        out_shape=jax.ShapeDtypeStruct((n, n_out), jnp.float32),
    )(sp, g, dis, b2, wf1, bf1, wf2, bf2)


def kernel(x, edge_index, W1, b1, W2, b2, Wf1, bf1, Wf2, bf2):
    n, d = x.shape
    e = edge_index.shape[1]
    ew = e // _NW
    nch = ew // _K
    idx_rs = edge_index.reshape(2, _NW, nch, _K).transpose(1, 2, 0, 3)

    zeros_n = jnp.zeros((n,), jnp.float32)
    ones_k = jnp.ones((_K,), jnp.float32)
    zeros_rt = jnp.zeros(((n // _NS) // 8 * 8, d), jnp.float32)

    bsz = 2000

    degp = _sc_deg(idx_rs, zeros_n, ones_k, n)
    degT = degp.reshape(_NC, n).T
    g1, dis = _tc_g1(degT, x, W1, bsz)
    s1p = _sc_edges(g1, idx_rs, zeros_rt)
    g2 = _tc_layer(s1p, g1, dis, b1.reshape(1, d), W2, bsz)
    s2p = _sc_edges(g2, idx_rs, zeros_rt)
    out = _tc_head(s2p, g2, dis, b2.reshape(1, d), Wf1,
                   bf1.reshape(1, -1), Wf2, bf2.reshape(1, -1), bsz)
    return out
